# Initial kernel scaffold; baseline (speedup 1.0000x reference)
import jax
import jax.numpy as jnp
from jax.experimental import pallas as pl


def kernel(H, X_node):
    raise NotImplementedError("write your pallas kernel here")


# speedup vs baseline: 4.4838x; 4.4838x over previous
import functools

import jax
import jax.numpy as jnp
from jax import lax
from jax.experimental import pallas as pl
from jax.experimental.pallas import tpu as pltpu
from jax.experimental.pallas import tpu_sc as plsc

V = 10000
E = 320000
D = 128

NC = 2
NS = 16
NW = NC * NS

R = 80
NBLK = E // R
BPW = NBLK // NW
ROWS_PW = BPW * R
V_PAD = 10240
VPT = V_PAD // NS


def _sc_partials(H, X_blocks, Z):
    @functools.partial(
        pl.kernel,
        out_type=jax.ShapeDtypeStruct((NC, V_PAD, D), jnp.float32),
        mesh=plsc.VectorSubcoreMesh(
            core_axis_name="c", subcore_axis_name="s"),
        scratch_types=[
            pltpu.VMEM((BPW, R), jnp.int32),
            pltpu.VMEM((R, D), jnp.float32),
            pltpu.VMEM_SHARED((V_PAD, D), jnp.float32),
        ],
    )
    def k(h_hbm, x_hbm, z_hbm, part_hbm, idx_v, rows_v, acc_s):
        cid = lax.axis_index("c")
        sid = lax.axis_index("s")
        wid = sid * NC + cid

        pltpu.sync_copy(z_hbm, acc_s.at[pl.ds(sid * VPT, VPT)])
        pltpu.sync_copy(x_hbm.at[wid], idx_v)
        plsc.subcore_barrier()

        def body(t, carry):
            e0 = wid * ROWS_PW + t * R
            pltpu.sync_copy(h_hbm.at[pl.ds(e0, R)], rows_v)
            pltpu.sync_copy(rows_v, acc_s.at[idx_v.at[t]], add=True)
            return carry

        lax.fori_loop(0, BPW, body, 0)
        plsc.subcore_barrier()

        pltpu.sync_copy(
            acc_s.at[pl.ds(sid * VPT, VPT)],
            part_hbm.at[cid, pl.ds(sid * VPT, VPT)],
        )

    return k(H, X_blocks, Z)


def _combine(parts):
    bv = 2048

    def body(p_ref, o_ref):
        o_ref[...] = p_ref[0] + p_ref[1]

    return pl.pallas_call(
        body,
        grid=(V_PAD // bv,),
        in_specs=[pl.BlockSpec((NC, bv, D), lambda i: (0, i, 0))],
        out_specs=pl.BlockSpec((bv, D), lambda i: (i, 0)),
        out_shape=jax.ShapeDtypeStruct((V_PAD, D), jnp.float32),
    )(parts)


def kernel(H, X_node):
    X_blocks = X_node.astype(jnp.int32).reshape(NW, BPW, R)
    Z = jnp.zeros((VPT, D), jnp.float32)
    parts = _sc_partials(H, X_blocks, Z)
    return _combine(parts)[:V]


# speedup vs baseline: 7.0099x; 1.5634x over previous
import functools

import jax
import jax.numpy as jnp
from jax import lax
from jax.experimental import pallas as pl
from jax.experimental.pallas import tpu as pltpu
from jax.experimental.pallas import tpu_sc as plsc

V = 10000
E = 320000
D = 128

NC = 2
NS = 16
NW = NC * NS

R = 80
NBLK = E // R
BPW = NBLK // NW
ROWS_PW = BPW * R
V_PAD = 10240
VPT = V_PAD // NS


def _sc_partials(H, X_blocks, Z):
    @functools.partial(
        pl.kernel,
        out_type=jax.ShapeDtypeStruct((NC, V_PAD, D), jnp.float32),
        mesh=plsc.VectorSubcoreMesh(
            core_axis_name="c", subcore_axis_name="s"),
        scratch_types=[
            pltpu.VMEM((BPW, R), jnp.int32),
            pltpu.VMEM((R, D), jnp.float32),
            pltpu.VMEM((R, D), jnp.float32),
            pltpu.VMEM_SHARED((V_PAD, D), jnp.float32),
            pltpu.SemaphoreType.DMA,
            pltpu.SemaphoreType.DMA,
        ],
    )
    def k(h_hbm, x_hbm, z_hbm, part_hbm, idx_v, rows0, rows1, acc_s,
          sem0, sem1):
        cid = lax.axis_index("c")
        sid = lax.axis_index("s")
        wid = sid * NC + cid
        wbase = wid * ROWS_PW

        pltpu.async_copy(h_hbm.at[pl.ds(wbase, R)], rows0, sem0)
        pltpu.async_copy(h_hbm.at[pl.ds(wbase + R, R)], rows1, sem1)
        pltpu.sync_copy(z_hbm, acc_s.at[pl.ds(sid * VPT, VPT)])
        pltpu.sync_copy(x_hbm.at[wid], idx_v)
        plsc.subcore_barrier()

        def body(t, carry):
            for b, (buf, sem) in enumerate(((rows0, sem0), (rows1, sem1))):
                blk = 2 * t + b

                @pl.when(blk < BPW)
                def _():
                    pltpu.make_async_copy(
                        h_hbm.at[pl.ds(0, R)], buf, sem).wait()
                    pltpu.sync_copy(buf, acc_s.at[idx_v.at[blk]], add=True)

                    @pl.when(blk + 2 < BPW)
                    def _():
                        e0 = wbase + (blk + 2) * R
                        pltpu.async_copy(h_hbm.at[pl.ds(e0, R)], buf, sem)

            return carry

        lax.fori_loop(0, (BPW + 1) // 2, body, 0)
        plsc.subcore_barrier()

        pltpu.sync_copy(
            acc_s.at[pl.ds(sid * VPT, VPT)],
            part_hbm.at[cid, pl.ds(sid * VPT, VPT)],
        )

    return k(H, X_blocks, Z)


def _combine(parts):
    bv = 2048

    def body(p_ref, o_ref):
        o_ref[...] = p_ref[0] + p_ref[1]

    return pl.pallas_call(
        body,
        grid=(V_PAD // bv,),
        in_specs=[pl.BlockSpec((NC, bv, D), lambda i: (0, i, 0))],
        out_specs=pl.BlockSpec((bv, D), lambda i: (i, 0)),
        out_shape=jax.ShapeDtypeStruct((V_PAD, D), jnp.float32),
    )(parts)


def kernel(H, X_node):
    X_blocks = X_node.astype(jnp.int32).reshape(NW, BPW, R)
    Z = jnp.zeros((VPT, D), jnp.float32)
    parts = _sc_partials(H, X_blocks, Z)
    return _combine(parts)[:V]


# speedup vs baseline: 7.2084x; 1.0283x over previous
import functools

import jax
import jax.numpy as jnp
from jax import lax
from jax.experimental import pallas as pl
from jax.experimental.pallas import tpu as pltpu
from jax.experimental.pallas import tpu_sc as plsc

V = 10000
E = 320000
D = 128

NC = 2
NS = 16
NW = NC * NS

R = 80
NBLK = E // R
BPW = NBLK // NW
ROWS_PW = BPW * R
V_PAD = 10240
VPT = V_PAD // NS


def _sc_partials(H, X_blocks, Z):
    @functools.partial(
        pl.kernel,
        out_type=jax.ShapeDtypeStruct((NC, V_PAD, D), jnp.float32),
        mesh=plsc.VectorSubcoreMesh(
            core_axis_name="c", subcore_axis_name="s"),
        scratch_types=[
            pltpu.VMEM((BPW, R), jnp.int32),
            [pltpu.VMEM((R, D), jnp.float32)] * 3,
            pltpu.VMEM_SHARED((V_PAD, D), jnp.float32),
            [pltpu.SemaphoreType.DMA] * 3,
            [pltpu.SemaphoreType.DMA] * 3,
        ],
    )
    def k(h_hbm, x_hbm, z_hbm, part_hbm, idx_v, bufs, acc_s, sem_g, sem_s):
        cid = lax.axis_index("c")
        sid = lax.axis_index("s")
        wid = sid * NC + cid
        wbase = wid * ROWS_PW

        for j in range(3):
            pltpu.async_copy(
                h_hbm.at[pl.ds(wbase + j * R, R)], bufs[j], sem_g[j])
        pltpu.sync_copy(z_hbm, acc_s.at[pl.ds(sid * VPT, VPT)])
        pltpu.sync_copy(x_hbm.at[wid], idx_v)
        plsc.subcore_barrier()

        def body(t, carry):
            for b in range(3):
                blk = 3 * t + b
                m = (b + 2) % 3

                @pl.when(blk < BPW)
                def _():
                    pltpu.make_async_copy(
                        h_hbm.at[pl.ds(0, R)], bufs[b], sem_g[b]).wait()
                    pltpu.async_copy(
                        bufs[b], acc_s.at[idx_v.at[blk]], sem_s[b], add=True)

                    @pl.when((blk >= 1) & (blk + 2 < BPW))
                    def _():
                        pltpu.make_async_copy(
                            bufs[m], acc_s.at[idx_v.at[0]], sem_s[m]).wait()
                        e0 = wbase + (blk + 2) * R
                        pltpu.async_copy(
                            h_hbm.at[pl.ds(e0, R)], bufs[m], sem_g[m])

            return carry

        lax.fori_loop(0, (BPW + 2) // 3, body, 0)
        for j in range(3):
            pltpu.make_async_copy(
                bufs[j], acc_s.at[idx_v.at[0]], sem_s[j]).wait()
        plsc.subcore_barrier()

        pltpu.sync_copy(
            acc_s.at[pl.ds(sid * VPT, VPT)],
            part_hbm.at[cid, pl.ds(sid * VPT, VPT)],
        )

    return k(H, X_blocks, Z)


def _combine(parts):
    bv = 2048

    def body(p_ref, o_ref):
        o_ref[...] = p_ref[0] + p_ref[1]

    return pl.pallas_call(
        body,
        grid=(V_PAD // bv,),
        in_specs=[pl.BlockSpec((NC, bv, D), lambda i: (0, i, 0))],
        out_specs=pl.BlockSpec((bv, D), lambda i: (i, 0)),
        out_shape=jax.ShapeDtypeStruct((V_PAD, D), jnp.float32),
    )(parts)


def kernel(H, X_node):
    X_blocks = X_node.astype(jnp.int32).reshape(NW, BPW, R)
    Z = jnp.zeros((VPT, D), jnp.float32)
    parts = _sc_partials(H, X_blocks, Z)
    return _combine(parts)[:V]


# speedup vs baseline: 7.3759x; 1.0232x over previous
import functools

import jax
import jax.numpy as jnp
from jax import lax
from jax.experimental import pallas as pl
from jax.experimental.pallas import tpu as pltpu
from jax.experimental.pallas import tpu_sc as plsc

V = 10000
E = 320000
D = 128

NC = 2
NS = 16
NW = NC * NS

R = 80
NBLK = E // R
BPW = NBLK // NW
ROWS_PW = BPW * R
V_PAD = 10240
VPT = V_PAD // NS


def _sc_partials(H, X_blocks, Z):
    @functools.partial(
        pl.kernel,
        out_type=jax.ShapeDtypeStruct((NC, V_PAD, D), jnp.float32),
        mesh=plsc.VectorSubcoreMesh(
            core_axis_name="c", subcore_axis_name="s"),
        scratch_types=[
            pltpu.VMEM((BPW, R), jnp.int32),
            [pltpu.VMEM((R, D), jnp.float32)] * 3,
            pltpu.VMEM_SHARED((V_PAD, D), jnp.float32),
            [pltpu.SemaphoreType.DMA] * 3,
            [pltpu.SemaphoreType.DMA] * 3,
        ],
    )
    def k(h_hbm, x_hbm, z_hbm, part_hbm, idx_v, bufs, acc_s, sem_g, sem_s):
        cid = lax.axis_index("c")
        sid = lax.axis_index("s")
        wid = sid * NC + cid
        wbase = wid * ROWS_PW

        for j in range(3):
            pltpu.async_copy(
                h_hbm.at[pl.ds(wbase + j * R, R)], bufs[j], sem_g[j])
        pltpu.sync_copy(z_hbm, acc_s.at[pl.ds(sid * VPT, VPT)])
        pltpu.sync_copy(x_hbm.at[wid], idx_v)
        plsc.subcore_barrier()

        def body(t, carry):
            for b in range(3):
                blk = 3 * t + b
                m = (b + 2) % 3

                @pl.when(blk < BPW)
                def _():
                    pltpu.make_async_copy(
                        h_hbm.at[pl.ds(0, R)], bufs[b], sem_g[b]).wait()
                    pltpu.async_copy(
                        bufs[b], acc_s.at[idx_v.at[blk]], sem_s[b], add=True)

                    @pl.when((blk >= 1) & (blk + 2 < BPW))
                    def _():
                        pltpu.make_async_copy(
                            bufs[m], acc_s.at[idx_v.at[0]], sem_s[m]).wait()
                        e0 = wbase + (blk + 2) * R
                        pltpu.async_copy(
                            h_hbm.at[pl.ds(e0, R)], bufs[m], sem_g[m])

            return carry

        lax.fori_loop(0, (BPW + 2) // 3, body, 0)
        for j in range(3):
            pltpu.make_async_copy(
                bufs[j], acc_s.at[idx_v.at[0]], sem_s[j]).wait()
        plsc.subcore_barrier()

        pltpu.sync_copy(
            acc_s.at[pl.ds(sid * VPT, VPT)],
            part_hbm.at[cid, pl.ds(sid * VPT, VPT)],
        )

    return k(H, X_blocks, Z)


def _combine(parts):
    bv = 2048

    def body(p_ref, o_ref):
        o_ref[...] = p_ref[0] + p_ref[1]

    return pl.pallas_call(
        body,
        grid=(V // bv,),
        in_specs=[pl.BlockSpec((NC, bv, D), lambda i: (0, i, 0))],
        out_specs=pl.BlockSpec((bv, D), lambda i: (i, 0)),
        out_shape=jax.ShapeDtypeStruct((V, D), jnp.float32),
    )(parts)


def kernel(H, X_node):
    X_blocks = X_node.astype(jnp.int32).reshape(NW, BPW, R)
    Z = jnp.zeros((VPT, D), jnp.float32)
    parts = _sc_partials(H, X_blocks, Z)
    return _combine(parts)


# speedup vs baseline: 7.3761x; 1.0000x over previous
import functools

import jax
import jax.numpy as jnp
from jax import lax
from jax.experimental import pallas as pl
from jax.experimental.pallas import tpu as pltpu
from jax.experimental.pallas import tpu_sc as plsc

V = 10000
E = 320000
D = 128

NC = 2
NS = 16
NW = NC * NS

R = 80
NBLK = E // R
BPW = NBLK // NW
ROWS_PW = BPW * R
V_PAD = 10240
VPT = V_PAD // NS


def _sc_partials(H, X_blocks, Z):
    @functools.partial(
        pl.kernel,
        out_type=jax.ShapeDtypeStruct((NC, V, D), jnp.float32),
        mesh=plsc.VectorSubcoreMesh(
            core_axis_name="c", subcore_axis_name="s"),
        scratch_types=[
            pltpu.VMEM((BPW, R), jnp.int32),
            [pltpu.VMEM((R, D), jnp.float32)] * 3,
            pltpu.VMEM_SHARED((V_PAD, D), jnp.float32),
            [pltpu.SemaphoreType.DMA] * 3,
            [pltpu.SemaphoreType.DMA] * 3,
        ],
    )
    def k(h_hbm, x_hbm, z_hbm, part_hbm, idx_v, bufs, acc_s, sem_g, sem_s):
        cid = lax.axis_index("c")
        sid = lax.axis_index("s")
        wid = sid * NC + cid
        wbase = wid * ROWS_PW

        for j in range(3):
            pltpu.async_copy(
                h_hbm.at[pl.ds(wbase + j * R, R)], bufs[j], sem_g[j])
        pltpu.sync_copy(z_hbm, acc_s.at[pl.ds(sid * VPT, VPT)])
        pltpu.sync_copy(x_hbm.at[wid], idx_v)
        plsc.subcore_barrier()

        def body(t, carry):
            for b in range(3):
                blk = 3 * t + b
                m = (b + 2) % 3

                @pl.when(blk < BPW)
                def _():
                    pltpu.make_async_copy(
                        h_hbm.at[pl.ds(0, R)], bufs[b], sem_g[b]).wait()
                    pltpu.async_copy(
                        bufs[b], acc_s.at[idx_v.at[blk]], sem_s[b], add=True)

                    @pl.when((blk >= 1) & (blk + 2 < BPW))
                    def _():
                        pltpu.make_async_copy(
                            bufs[m], acc_s.at[idx_v.at[0]], sem_s[m]).wait()
                        e0 = wbase + (blk + 2) * R
                        pltpu.async_copy(
                            h_hbm.at[pl.ds(e0, R)], bufs[m], sem_g[m])

            return carry

        lax.fori_loop(0, (BPW + 2) // 3, body, 0)
        for j in range(3):
            pltpu.make_async_copy(
                bufs[j], acc_s.at[idx_v.at[0]], sem_s[j]).wait()
        plsc.subcore_barrier()

        @pl.when(sid < NS - 1)
        def _():
            pltpu.sync_copy(
                acc_s.at[pl.ds(sid * VPT, VPT)],
                part_hbm.at[cid, pl.ds(sid * VPT, VPT)],
            )

        @pl.when(sid == NS - 1)
        def _():
            last = V - (NS - 1) * VPT
            pltpu.sync_copy(
                acc_s.at[pl.ds((NS - 1) * VPT, last)],
                part_hbm.at[cid, pl.ds((NS - 1) * VPT, last)],
            )

    return k(H, X_blocks, Z)


def _combine(parts):
    bv = 2000

    def body(p_ref, o_ref):
        o_ref[...] = p_ref[0] + p_ref[1]

    return pl.pallas_call(
        body,
        grid=(V // bv,),
        in_specs=[pl.BlockSpec((NC, bv, D), lambda i: (0, i, 0))],
        out_specs=pl.BlockSpec((bv, D), lambda i: (i, 0)),
        out_shape=jax.ShapeDtypeStruct((V, D), jnp.float32),
    )(parts)


def kernel(H, X_node):
    X_blocks = X_node.astype(jnp.int32).reshape(NW, BPW, R)
    Z = jnp.zeros((VPT, D), jnp.float32)
    parts = _sc_partials(H, X_blocks, Z)
    return _combine(parts)


# speedup vs baseline: 7.4521x; 1.0103x over previous
import functools

import jax
import jax.numpy as jnp
from jax import lax
from jax.experimental import pallas as pl
from jax.experimental.pallas import tpu as pltpu
from jax.experimental.pallas import tpu_sc as plsc

V = 10000
E = 320000
D = 128

NC = 2
NS = 16
NW = NC * NS

R = 80
NBLK = E // R
BPW = NBLK // NW
ROWS_PW = BPW * R
V_PAD = 10240
VPT = V_PAD // NS


def _sc_partials(H, X_blocks, Z):
    @functools.partial(
        pl.kernel,
        out_type=jax.ShapeDtypeStruct((NC, V, D), jnp.float32),
        mesh=plsc.VectorSubcoreMesh(
            core_axis_name="c", subcore_axis_name="s"),
        scratch_types=[
            pltpu.VMEM((BPW, R), jnp.int32),
            [pltpu.VMEM((R, D), jnp.float32)] * 3,
            pltpu.VMEM_SHARED((V_PAD, D), jnp.float32),
            [pltpu.SemaphoreType.DMA] * 3,
            [pltpu.SemaphoreType.DMA] * 3,
        ],
    )
    def k(h_hbm, x_hbm, z_hbm, part_hbm, idx_v, bufs, acc_s, sem_g, sem_s):
        cid = lax.axis_index("c")
        sid = lax.axis_index("s")
        wid = sid * NC + cid
        wbase = wid * ROWS_PW

        for j in range(3):
            pltpu.async_copy(
                h_hbm.at[pl.ds(wbase + j * R, R)], bufs[j], sem_g[j])
        pltpu.sync_copy(z_hbm, acc_s.at[pl.ds(sid * VPT, VPT)])
        pltpu.sync_copy(x_hbm.at[wid], idx_v)
        plsc.subcore_barrier()

        def body(t, carry):
            for b in range(3):
                blk = 3 * t + b
                m = (b + 2) % 3

                @pl.when(blk < BPW)
                def _():
                    pltpu.make_async_copy(
                        h_hbm.at[pl.ds(0, R)], bufs[b], sem_g[b]).wait()
                    pltpu.async_copy(
                        bufs[b], acc_s.at[idx_v.at[blk]], sem_s[b], add=True)

                    @pl.when((blk >= 1) & (blk + 2 < BPW))
                    def _():
                        pltpu.make_async_copy(
                            bufs[m], acc_s.at[idx_v.at[0]], sem_s[m]).wait()
                        e0 = wbase + (blk + 2) * R
                        pltpu.async_copy(
                            h_hbm.at[pl.ds(e0, R)], bufs[m], sem_g[m])

            return carry

        lax.fori_loop(0, (BPW + 2) // 3, body, 0)
        for j in range(3):
            pltpu.make_async_copy(
                bufs[j], acc_s.at[idx_v.at[0]], sem_s[j]).wait()
        plsc.subcore_barrier()

        @pl.when(sid < NS - 1)
        def _():
            pltpu.sync_copy(
                acc_s.at[pl.ds(sid * VPT, VPT)],
                part_hbm.at[cid, pl.ds(sid * VPT, VPT)],
            )

        @pl.when(sid == NS - 1)
        def _():
            last = V - (NS - 1) * VPT
            pltpu.sync_copy(
                acc_s.at[pl.ds((NS - 1) * VPT, last)],
                part_hbm.at[cid, pl.ds((NS - 1) * VPT, last)],
            )

    return k(H, X_blocks, Z)


def _combine(parts):
    bv = 5000

    def body(p_ref, o_ref):
        o_ref[...] = p_ref[0] + p_ref[1]

    return pl.pallas_call(
        body,
        grid=(V // bv,),
        in_specs=[pl.BlockSpec((NC, bv, D), lambda i: (0, i, 0))],
        out_specs=pl.BlockSpec((bv, D), lambda i: (i, 0)),
        out_shape=jax.ShapeDtypeStruct((V, D), jnp.float32),
    )(parts)


def kernel(H, X_node):
    X_blocks = X_node.astype(jnp.int32).reshape(NW, BPW, R)
    Z = jnp.zeros((VPT, D), jnp.float32)
    parts = _sc_partials(H, X_blocks, Z)
    return _combine(parts)


# speedup vs baseline: 7.4542x; 1.0003x over previous
import functools

import jax
import jax.numpy as jnp
from jax import lax
from jax.experimental import pallas as pl
from jax.experimental.pallas import tpu as pltpu
from jax.experimental.pallas import tpu_sc as plsc

V = 10000
E = 320000
D = 128

NC = 2
NS = 16
NW = NC * NS

R = 80
NBLK = E // R
BPW = NBLK // NW
ROWS_PW = BPW * R
V_PAD = 10240
VPT = V_PAD // NS


def _sc_partials(H, X_blocks, Z):
    @functools.partial(
        pl.kernel,
        out_type=jax.ShapeDtypeStruct((NC, V, D), jnp.float32),
        mesh=plsc.VectorSubcoreMesh(
            core_axis_name="c", subcore_axis_name="s"),
        scratch_types=[
            pltpu.VMEM((BPW, R), jnp.int32),
            [pltpu.VMEM((R, D), jnp.float32)] * 3,
            pltpu.VMEM_SHARED((V_PAD, D), jnp.float32),
            [pltpu.SemaphoreType.DMA] * 3,
            [pltpu.SemaphoreType.DMA] * 3,
        ],
    )
    def k(h_hbm, x_hbm, z_hbm, part_hbm, idx_v, bufs, acc_s, sem_g, sem_s):
        cid = lax.axis_index("c")
        sid = lax.axis_index("s")
        wid = cid * NS + sid
        wbase = wid * ROWS_PW

        for j in range(3):
            pltpu.async_copy(
                h_hbm.at[pl.ds(wbase + j * R, R)], bufs[j], sem_g[j])
        pltpu.sync_copy(z_hbm, acc_s.at[pl.ds(sid * VPT, VPT)])
        pltpu.sync_copy(x_hbm.at[wid], idx_v)
        plsc.subcore_barrier()

        def body(t, carry):
            for b in range(3):
                blk = 3 * t + b
                m = (b + 2) % 3

                @pl.when(blk < BPW)
                def _():
                    pltpu.make_async_copy(
                        h_hbm.at[pl.ds(0, R)], bufs[b], sem_g[b]).wait()
                    pltpu.async_copy(
                        bufs[b], acc_s.at[idx_v.at[blk]], sem_s[b], add=True)

                    @pl.when((blk >= 1) & (blk + 2 < BPW))
                    def _():
                        pltpu.make_async_copy(
                            bufs[m], acc_s.at[idx_v.at[0]], sem_s[m]).wait()
                        e0 = wbase + (blk + 2) * R
                        pltpu.async_copy(
                            h_hbm.at[pl.ds(e0, R)], bufs[m], sem_g[m])

            return carry

        lax.fori_loop(0, (BPW + 2) // 3, body, 0)
        for j in range(3):
            pltpu.make_async_copy(
                bufs[j], acc_s.at[idx_v.at[0]], sem_s[j]).wait()
        plsc.subcore_barrier()

        @pl.when(sid < NS - 1)
        def _():
            pltpu.sync_copy(
                acc_s.at[pl.ds(sid * VPT, VPT)],
                part_hbm.at[cid, pl.ds(sid * VPT, VPT)],
            )

        @pl.when(sid == NS - 1)
        def _():
            last = V - (NS - 1) * VPT
            pltpu.sync_copy(
                acc_s.at[pl.ds((NS - 1) * VPT, last)],
                part_hbm.at[cid, pl.ds((NS - 1) * VPT, last)],
            )

    return k(H, X_blocks, Z)


def _combine(parts):
    bv = 5000

    def body(p_ref, o_ref):
        o_ref[...] = p_ref[0] + p_ref[1]

    return pl.pallas_call(
        body,
        grid=(V // bv,),
        in_specs=[pl.BlockSpec((NC, bv, D), lambda i: (0, i, 0))],
        out_specs=pl.BlockSpec((bv, D), lambda i: (i, 0)),
        out_shape=jax.ShapeDtypeStruct((V, D), jnp.float32),
    )(parts)


def kernel(H, X_node):
    X_blocks = X_node.astype(jnp.int32).reshape(NW, BPW, R)
    Z = jnp.zeros((VPT, D), jnp.float32)
    parts = _sc_partials(H, X_blocks, Z)
    return _combine(parts)
